# baseline (speedup 1.0000x reference)
import jax
import jax.numpy as jnp
from jax.experimental import pallas as pl
from jax.experimental.pallas import tpu as pltpu

_CHUNK_ROWS = 512
_K = 6


def _copy_body(img_in, txt_in, img_out, txt_out, buf_i, buf_t, ld_i, ld_t, st_i, st_t):
    n_chunks = img_in.shape[0] // _CHUNK_ROWS

    streams = []
    for src, dst, buf, ld_sem, st_sem in (
        (img_in, img_out, buf_i, ld_i, st_i),
        (txt_in, txt_out, buf_t, ld_t, st_t),
    ):
        lds, sts = [], []
        for c in range(n_chunks):
            s = c % _K
            sl = pl.ds(c * _CHUNK_ROWS, _CHUNK_ROWS)
            lds.append(pltpu.make_async_copy(src.at[sl], buf.at[s], ld_sem.at[s]))
            sts.append(pltpu.make_async_copy(buf.at[s], dst.at[sl], st_sem.at[s]))
        streams.append((lds, sts))

    for k in range(_K):
        for lds, _ in streams:
            lds[k].start()
    for k in range(n_chunks):
        for lds, sts in streams:
            lds[k].wait()
            sts[k].start()
        if k + _K < n_chunks:
            for lds, sts in streams:
                sts[k].wait()
                lds[k + _K].start()
    for k in range(n_chunks - _K, n_chunks):
        for _, sts in streams:
            sts[k].wait()


def kernel(image_feat, text_feat):
    n_cols = image_feat.shape[1]
    out = pl.pallas_call(
        _copy_body,
        in_specs=[
            pl.BlockSpec(memory_space=pl.MemorySpace.ANY),
            pl.BlockSpec(memory_space=pl.MemorySpace.ANY),
        ],
        out_specs=[
            pl.BlockSpec(memory_space=pl.MemorySpace.ANY),
            pl.BlockSpec(memory_space=pl.MemorySpace.ANY),
        ],
        out_shape=[
            jax.ShapeDtypeStruct(image_feat.shape, image_feat.dtype),
            jax.ShapeDtypeStruct(text_feat.shape, text_feat.dtype),
        ],
        scratch_shapes=[
            pltpu.VMEM((_K, _CHUNK_ROWS, n_cols), jnp.float32),
            pltpu.VMEM((_K, _CHUNK_ROWS, n_cols), jnp.float32),
            pltpu.SemaphoreType.DMA((_K,)),
            pltpu.SemaphoreType.DMA((_K,)),
            pltpu.SemaphoreType.DMA((_K,)),
            pltpu.SemaphoreType.DMA((_K,)),
        ],
    )(image_feat, text_feat)
    return (out[0], out[1])


# speedup vs baseline: 1.0030x; 1.0030x over previous
import jax
import jax.numpy as jnp
from jax.experimental import pallas as pl
from jax.experimental.pallas import tpu as pltpu

_CHUNK_ROWS = 2048
_K = 3


def _copy_body(img_in, txt_in, img_out, txt_out, buf_i, buf_t, ld_i, ld_t, st_i, st_t):
    n_chunks = img_in.shape[0] // _CHUNK_ROWS

    streams = []
    for src, dst, buf, ld_sem, st_sem in (
        (img_in, img_out, buf_i, ld_i, st_i),
        (txt_in, txt_out, buf_t, ld_t, st_t),
    ):
        lds, sts = [], []
        for c in range(n_chunks):
            s = c % _K
            sl = pl.ds(c * _CHUNK_ROWS, _CHUNK_ROWS)
            lds.append(pltpu.make_async_copy(src.at[sl], buf.at[s], ld_sem.at[s]))
            sts.append(pltpu.make_async_copy(buf.at[s], dst.at[sl], st_sem.at[s]))
        streams.append((lds, sts))

    for k in range(_K):
        for lds, _ in streams:
            lds[k].start()
    for k in range(n_chunks):
        for lds, sts in streams:
            lds[k].wait()
            sts[k].start()
        if k + _K < n_chunks:
            for lds, sts in streams:
                sts[k].wait()
                lds[k + _K].start()
    for k in range(n_chunks - _K, n_chunks):
        for _, sts in streams:
            sts[k].wait()


def kernel(image_feat, text_feat):
    n_cols = image_feat.shape[1]
    out = pl.pallas_call(
        _copy_body,
        in_specs=[
            pl.BlockSpec(memory_space=pl.MemorySpace.ANY),
            pl.BlockSpec(memory_space=pl.MemorySpace.ANY),
        ],
        out_specs=[
            pl.BlockSpec(memory_space=pl.MemorySpace.ANY),
            pl.BlockSpec(memory_space=pl.MemorySpace.ANY),
        ],
        out_shape=[
            jax.ShapeDtypeStruct(image_feat.shape, image_feat.dtype),
            jax.ShapeDtypeStruct(text_feat.shape, text_feat.dtype),
        ],
        scratch_shapes=[
            pltpu.VMEM((_K, _CHUNK_ROWS, n_cols), jnp.float32),
            pltpu.VMEM((_K, _CHUNK_ROWS, n_cols), jnp.float32),
            pltpu.SemaphoreType.DMA((_K,)),
            pltpu.SemaphoreType.DMA((_K,)),
            pltpu.SemaphoreType.DMA((_K,)),
            pltpu.SemaphoreType.DMA((_K,)),
        ],
    )(image_feat, text_feat)
    return (out[0], out[1])
